# baseline (speedup 1.0000x reference)
import functools

import jax
import jax.numpy as jnp
from jax.experimental import pallas as pl
from jax.experimental.pallas import tpu as pltpu


def _nbeats_kernel(x_ref, w1_ref, b1_ref, w2_ref, b2_ref, w3_ref, b3_ref,
                   o_ref, *, nb, nf):
    inp = x_ref[...]
    out = b3_ref[...]
    for b in range(nb):
        h = jnp.dot(inp, w1_ref[b], preferred_element_type=jnp.float32)
        h = jnp.maximum(h + b1_ref[b], 0.0).astype(jnp.bfloat16)
        h = jnp.dot(h, w2_ref[b], preferred_element_type=jnp.float32)
        h = jnp.maximum(h + b2_ref[b], 0.0).astype(jnp.bfloat16)
        out = out + jnp.dot(h, w3_ref[b], preferred_element_type=jnp.float32)
    o_ref[...] = out


def kernel(x, w1, b1, w2, b2, w3, b3):
    B, t_in, nf = x.shape
    nb, _, hid = w1.shape
    t_out = w3.shape[-1]

    x2 = x[:, :, -1].astype(jnp.bfloat16)
    w1b = w1.astype(jnp.bfloat16)
    w2b = w2.astype(jnp.bfloat16)
    w3b = w3.astype(jnp.bfloat16)
    b1r = b1.reshape(nb, 1, hid)
    b2r = b2.reshape(nb, 1, hid)
    b3s = b3.sum(axis=0, keepdims=True)

    tb = 2048 if B % 2048 == 0 else B
    return pl.pallas_call(
        functools.partial(_nbeats_kernel, nb=nb, nf=nf),
        out_shape=jax.ShapeDtypeStruct((B, t_out), jnp.float32),
        grid=(B // tb,),
        in_specs=[
            pl.BlockSpec((tb, t_in), lambda i: (i, 0)),
            pl.BlockSpec(w1b.shape, lambda i: (0, 0, 0)),
            pl.BlockSpec(b1r.shape, lambda i: (0, 0, 0)),
            pl.BlockSpec(w2b.shape, lambda i: (0, 0, 0)),
            pl.BlockSpec(b2r.shape, lambda i: (0, 0, 0)),
            pl.BlockSpec(w3b.shape, lambda i: (0, 0, 0)),
            pl.BlockSpec(b3s.shape, lambda i: (0, 0)),
        ],
        out_specs=pl.BlockSpec((tb, t_out), lambda i: (i, 0)),
        compiler_params=pltpu.CompilerParams(
            dimension_semantics=("parallel",)),
    )(x2, w1b, b1r, w2b, b2r, w3b, b3s)


# speedup vs baseline: 1.1004x; 1.1004x over previous
import functools

import jax
import jax.numpy as jnp
from jax.experimental import pallas as pl
from jax.experimental.pallas import tpu as pltpu


def _nbeats_kernel(x_ref, w1_ref, b1_ref, w2_ref, b2_ref, w3_ref, b3_ref,
                   o_ref, *, nb):
    inp = x_ref[...]
    out = jnp.sum(b3_ref[...], axis=0, keepdims=True)
    for b in range(nb):
        h = jnp.dot(inp, w1_ref[b].astype(jnp.bfloat16),
                    preferred_element_type=jnp.float32)
        h = jnp.maximum(h + b1_ref[b:b + 1, :], 0.0).astype(jnp.bfloat16)
        h = jnp.dot(h, w2_ref[b].astype(jnp.bfloat16),
                    preferred_element_type=jnp.float32)
        h = jnp.maximum(h + b2_ref[b:b + 1, :], 0.0).astype(jnp.bfloat16)
        out = out + jnp.dot(h, w3_ref[b].astype(jnp.bfloat16),
                            preferred_element_type=jnp.float32)
    o_ref[...] = out


def kernel(x, w1, b1, w2, b2, w3, b3):
    B, t_in, nf = x.shape
    nb, _, hid = w1.shape
    t_out = w3.shape[-1]

    inp = x[:, :, -1].astype(jnp.bfloat16)

    tb = 1024 if B % 1024 == 0 else B
    return pl.pallas_call(
        functools.partial(_nbeats_kernel, nb=nb),
        out_shape=jax.ShapeDtypeStruct((B, t_out), jnp.float32),
        grid=(B // tb,),
        in_specs=[
            pl.BlockSpec((tb, t_in), lambda i: (i, 0)),
            pl.BlockSpec(w1.shape, lambda i: (0, 0, 0)),
            pl.BlockSpec(b1.shape, lambda i: (0, 0)),
            pl.BlockSpec(w2.shape, lambda i: (0, 0, 0)),
            pl.BlockSpec(b2.shape, lambda i: (0, 0)),
            pl.BlockSpec(w3.shape, lambda i: (0, 0, 0)),
            pl.BlockSpec(b3.shape, lambda i: (0, 0)),
        ],
        out_specs=pl.BlockSpec((tb, t_out), lambda i: (i, 0)),
        compiler_params=pltpu.CompilerParams(
            dimension_semantics=("parallel",)),
    )(inp, w1, b1, w2, b2, w3, b3)


# speedup vs baseline: 1.1153x; 1.0135x over previous
import functools

import jax
import jax.numpy as jnp
from jax.experimental import pallas as pl
from jax.experimental.pallas import tpu as pltpu


def _nbeats_kernel(x_ref, w1_ref, b1_ref, w2_ref, b2_ref, w3_ref, b3_ref,
                   o_ref, *, nb):
    inp = x_ref[...]
    out = jnp.sum(b3_ref[...], axis=0, keepdims=True)
    for b in range(nb):
        h = jnp.dot(inp, w1_ref[b].astype(jnp.bfloat16),
                    preferred_element_type=jnp.float32)
        h = jnp.maximum(h + b1_ref[b:b + 1, :], 0.0).astype(jnp.bfloat16)
        h = jnp.dot(h, w2_ref[b].astype(jnp.bfloat16),
                    preferred_element_type=jnp.float32)
        h = jnp.maximum(h + b2_ref[b:b + 1, :], 0.0).astype(jnp.bfloat16)
        out = out + jnp.dot(h, w3_ref[b].astype(jnp.bfloat16),
                            preferred_element_type=jnp.float32)
    o_ref[...] = out


def kernel(x, w1, b1, w2, b2, w3, b3):
    B, t_in, nf = x.shape
    nb, _, hid = w1.shape
    t_out = w3.shape[-1]

    inp = x[:, :, -1].astype(jnp.bfloat16)

    tb = B
    return pl.pallas_call(
        functools.partial(_nbeats_kernel, nb=nb),
        out_shape=jax.ShapeDtypeStruct((B, t_out), jnp.float32),
        grid=(B // tb,),
        in_specs=[
            pl.BlockSpec((tb, t_in), lambda i: (i, 0)),
            pl.BlockSpec(w1.shape, lambda i: (0, 0, 0)),
            pl.BlockSpec(b1.shape, lambda i: (0, 0)),
            pl.BlockSpec(w2.shape, lambda i: (0, 0, 0)),
            pl.BlockSpec(b2.shape, lambda i: (0, 0)),
            pl.BlockSpec(w3.shape, lambda i: (0, 0, 0)),
            pl.BlockSpec(b3.shape, lambda i: (0, 0)),
        ],
        out_specs=pl.BlockSpec((tb, t_out), lambda i: (i, 0)),
        compiler_params=pltpu.CompilerParams(
            dimension_semantics=("parallel",)),
    )(inp, w1, b1, w2, b2, w3, b3)


# speedup vs baseline: 1.2190x; 1.0930x over previous
import functools

import jax
import jax.numpy as jnp
from jax.experimental import pallas as pl
from jax.experimental.pallas import tpu as pltpu


def _nbeats_kernel(x_ref, w1_ref, b1_ref, w2_ref, b2_ref, w3t_ref, b3_ref,
                   o_ref, *, nb):
    inp = x_ref[...]
    b3s = jnp.sum(b3_ref[...], axis=0, keepdims=True)
    out_t = jnp.swapaxes(b3s, 0, 1)
    for b in range(nb):
        h = jnp.dot(inp, w1_ref[b].astype(jnp.bfloat16),
                    preferred_element_type=jnp.float32)
        h = jnp.maximum(h + b1_ref[b:b + 1, :], 0.0).astype(jnp.bfloat16)
        h = jnp.dot(h, w2_ref[b].astype(jnp.bfloat16),
                    preferred_element_type=jnp.float32)
        h = jnp.maximum(h + b2_ref[b:b + 1, :], 0.0).astype(jnp.bfloat16)
        out_t = out_t + jnp.einsum("mk,nk->mn",
                                   w3t_ref[b].astype(jnp.bfloat16), h,
                                   preferred_element_type=jnp.float32)
    o_ref[...] = out_t


def kernel(x, w1, b1, w2, b2, w3, b3):
    B, t_in, nf = x.shape
    nb, _, hid = w1.shape
    t_out = w3.shape[-1]

    inp = x[:, :, -1].astype(jnp.bfloat16)
    w3t = jnp.swapaxes(w3, 1, 2)

    tb = 1024 if B % 1024 == 0 else B
    out_t = pl.pallas_call(
        functools.partial(_nbeats_kernel, nb=nb),
        out_shape=jax.ShapeDtypeStruct((t_out, B), jnp.float32),
        grid=(B // tb,),
        in_specs=[
            pl.BlockSpec((tb, t_in), lambda i: (i, 0)),
            pl.BlockSpec(w1.shape, lambda i: (0, 0, 0)),
            pl.BlockSpec(b1.shape, lambda i: (0, 0)),
            pl.BlockSpec(w2.shape, lambda i: (0, 0, 0)),
            pl.BlockSpec(b2.shape, lambda i: (0, 0)),
            pl.BlockSpec((nb, t_out, hid), lambda i: (0, 0, 0)),
            pl.BlockSpec(b3.shape, lambda i: (0, 0)),
        ],
        out_specs=pl.BlockSpec((t_out, tb), lambda i: (0, i)),
        compiler_params=pltpu.CompilerParams(
            dimension_semantics=("parallel",)),
    )(inp, w1, b1, w2, b2, w3t, b3)
    return out_t.T


# speedup vs baseline: 1.2416x; 1.0186x over previous
import functools

import jax
import jax.numpy as jnp
from jax.experimental import pallas as pl
from jax.experimental.pallas import tpu as pltpu


def _nbeats_kernel(x_ref, w1_ref, b1_ref, w2_ref, b2_ref, w3t_ref, b3_ref,
                   o_ref, *, nb):
    inp = x_ref[...]
    b3s = jnp.sum(b3_ref[...], axis=0, keepdims=True)
    out_t = jnp.swapaxes(b3s, 0, 1)
    for b in range(nb):
        h = jnp.dot(inp, w1_ref[b].astype(jnp.bfloat16),
                    preferred_element_type=jnp.float32).astype(jnp.bfloat16)
        h = jnp.maximum(h + b1_ref[b:b + 1, :].astype(jnp.bfloat16), 0)
        h = jnp.dot(h, w2_ref[b].astype(jnp.bfloat16),
                    preferred_element_type=jnp.float32).astype(jnp.bfloat16)
        h = jnp.maximum(h + b2_ref[b:b + 1, :].astype(jnp.bfloat16), 0)
        out_t = out_t + jnp.einsum("mk,nk->mn",
                                   w3t_ref[b].astype(jnp.bfloat16), h,
                                   preferred_element_type=jnp.float32)
    o_ref[...] = out_t


def kernel(x, w1, b1, w2, b2, w3, b3):
    B, t_in, nf = x.shape
    nb, _, hid = w1.shape
    t_out = w3.shape[-1]

    inp = x[:, :, -1].astype(jnp.bfloat16)
    w3t = jnp.swapaxes(w3, 1, 2)

    tb = B
    out_t = pl.pallas_call(
        functools.partial(_nbeats_kernel, nb=nb),
        out_shape=jax.ShapeDtypeStruct((t_out, B), jnp.float32),
        grid=(B // tb,),
        in_specs=[
            pl.BlockSpec((tb, t_in), lambda i: (i, 0)),
            pl.BlockSpec(w1.shape, lambda i: (0, 0, 0)),
            pl.BlockSpec(b1.shape, lambda i: (0, 0)),
            pl.BlockSpec(w2.shape, lambda i: (0, 0, 0)),
            pl.BlockSpec(b2.shape, lambda i: (0, 0)),
            pl.BlockSpec((nb, t_out, hid), lambda i: (0, 0, 0)),
            pl.BlockSpec(b3.shape, lambda i: (0, 0)),
        ],
        out_specs=pl.BlockSpec((t_out, tb), lambda i: (0, i)),
        compiler_params=pltpu.CompilerParams(
            dimension_semantics=("parallel",)),
    )(inp, w1, b1, w2, b2, w3t, b3)
    return out_t.T


# speedup vs baseline: 1.2559x; 1.0115x over previous
import functools

import jax
import jax.numpy as jnp
from jax.experimental import pallas as pl
from jax.experimental.pallas import tpu as pltpu


def _nbeats_kernel(x_ref, w1_ref, b1_ref, w2_ref, b2_ref, w3t_ref, b3_ref,
                   o_ref, *, nb):
    inp = x_ref[...]
    b3s = jnp.sum(b3_ref[...], axis=0, keepdims=True)
    out_t = jnp.swapaxes(b3s, 0, 1)
    for b in range(nb):
        h = jnp.dot(inp, w1_ref[b].astype(jnp.bfloat16),
                    preferred_element_type=jnp.float32)
        h = jnp.maximum(h + b1_ref[b:b + 1, :], 0.0).astype(jnp.bfloat16)
        h = jnp.dot(h, w2_ref[b].astype(jnp.bfloat16),
                    preferred_element_type=jnp.float32)
        h = jnp.maximum(h + b2_ref[b:b + 1, :], 0.0).astype(jnp.bfloat16)
        out_t = out_t + jnp.einsum("mk,nk->mn",
                                   w3t_ref[b].astype(jnp.bfloat16), h,
                                   preferred_element_type=jnp.float32)
    o_ref[...] = out_t


def kernel(x, w1, b1, w2, b2, w3, b3):
    B, t_in, nf = x.shape
    nb, _, hid = w1.shape
    t_out = w3.shape[-1]

    inp = x[:, :, -1].astype(jnp.bfloat16)
    w3t = jnp.swapaxes(w3, 1, 2)

    tb = B
    out_t = pl.pallas_call(
        functools.partial(_nbeats_kernel, nb=nb),
        out_shape=jax.ShapeDtypeStruct((t_out, B), jnp.float32),
        grid=(B // tb,),
        in_specs=[
            pl.BlockSpec((tb, t_in), lambda i: (i, 0)),
            pl.BlockSpec(w1.shape, lambda i: (0, 0, 0)),
            pl.BlockSpec(b1.shape, lambda i: (0, 0)),
            pl.BlockSpec(w2.shape, lambda i: (0, 0, 0)),
            pl.BlockSpec(b2.shape, lambda i: (0, 0)),
            pl.BlockSpec((nb, t_out, hid), lambda i: (0, 0, 0)),
            pl.BlockSpec(b3.shape, lambda i: (0, 0)),
        ],
        out_specs=pl.BlockSpec((t_out, tb), lambda i: (0, i)),
        compiler_params=pltpu.CompilerParams(
            dimension_semantics=("parallel",)),
    )(inp, w1, b1, w2, b2, w3t, b3)
    return out_t.T
